# baseline (speedup 1.0000x reference)
import functools

import jax
import jax.numpy as jnp
from jax import lax
from jax.experimental import pallas as pl
from jax.experimental.pallas import tpu as pltpu
from jax.experimental.pallas import tpu_sc as plsc

_NC = 2
_NS = 16
_NW = _NC * _NS
_L = 16


def _softplus_body(x_ref, o_ref):
    o_ref[...] = jax.nn.softplus(x_ref[...] - 1.0)


@functools.lru_cache(maxsize=None)
def _make_scatter(M, N, D):
    ppw = N // _NW
    blk = min(128, ppw)
    chunks = ppw // blk
    nbuf = 2
    assert chunks % nbuf == 0 and chunks // nbuf >= 2
    mesh = plsc.VectorSubcoreMesh(core_axis_name="c", subcore_axis_name="s",
                                  num_cores=_NC, num_subcores=_NS)

    @functools.partial(
        pl.kernel,
        mesh=mesh,
        compiler_params=pltpu.CompilerParams(needs_layout_passes=False),
        scratch_types=[
            [pltpu.VMEM((blk,), jnp.int32) for _ in range(nbuf)],
            [pltpu.VMEM((1, blk), jnp.float32) for _ in range(nbuf)],
            [pltpu.VMEM((D, blk), jnp.float32) for _ in range(nbuf)],
            [pltpu.VMEM((blk, 128), jnp.float32) for _ in range(nbuf)],
            [pltpu.SemaphoreType.DMA for _ in range(nbuf)],
            [pltpu.SemaphoreType.DMA for _ in range(nbuf)],
        ],
    )
    def scatter(idx_hbm, sp_hbm, vt_hbm, out_hbm,
                idx_vs, sp_vs, slab_vs, rows_vs, in_sems, out_sems):
        wid = lax.axis_index("s") * _NC + lax.axis_index("c")
        base = wid * ppw

        def load_descs(c, j):
            p0 = pl.multiple_of(base + c * blk, blk)
            r0 = p0 // blk
            return (
                pltpu.make_async_copy(idx_hbm.at[pl.ds(p0, blk)], idx_vs[j],
                                      in_sems[j]),
                pltpu.make_async_copy(sp_hbm.at[pl.ds(r0, 1), :], sp_vs[j],
                                      in_sems[j]),
                pltpu.make_async_copy(vt_hbm.at[:, pl.ds(p0, blk)], slab_vs[j],
                                      in_sems[j]),
            )

        def start_load(c, j):
            for d in load_descs(c, j):
                d.start()

        def wait_load(c, j):
            for d in load_descs(c, j):
                d.wait()

        def scatter_desc(j):
            return pltpu.make_async_copy(rows_vs[j], out_hbm.at[idx_vs[j]],
                                         out_sems[j])

        def assemble(j):
            rows_v = rows_vs[j]
            slab_v = slab_vs[j]
            sp_v = sp_vs[j]
            lanes = lax.iota(jnp.int32, _L)
            for g in range(blk // _L):
                rows_g = lanes + g * _L
                plsc.store_scatter(
                    rows_v, [rows_g, jnp.zeros((_L,), jnp.int32)],
                    sp_v[0, pl.ds(g * _L, _L)])
                for c in range(D):
                    plsc.store_scatter(
                        rows_v, [rows_g, jnp.full((_L,), c + 1, jnp.int32)],
                        slab_v[c, pl.ds(g * _L, _L)])

        for j in range(nbuf):
            start_load(j, j)

        def group(g, carry):
            c0 = g * nbuf
            for j in range(nbuf):
                wait_load(c0 + j, j)
                assemble(j)
                scatter_desc(j).start()
            nxt = c0 + nbuf

            @pl.when(nxt < chunks)
            def _():
                for j in range(nbuf):
                    scatter_desc(j).wait()
                    start_load(nxt + j, j)

            return carry

        lax.fori_loop(0, chunks // nbuf, group, 0)
        for j in range(nbuf):
            scatter_desc(j).wait()

    return scatter


def kernel(mem, idx, shape_raw, val):
    M, D = mem.shape
    N = idx.shape[0]
    sp2d = pl.pallas_call(
        _softplus_body,
        out_shape=jax.ShapeDtypeStruct((N // 128, 128), jnp.float32),
    )(shape_raw.reshape(N // 128, 128))
    vt = val.T
    out_ref = jax.new_ref(jnp.zeros((M, 128), jnp.float32))
    _make_scatter(M, N, D)(idx, sp2d, vt, out_ref)
    return out_ref[...][:, : D + 1]


# speedup vs baseline: 1.4136x; 1.4136x over previous
import functools

import jax
import jax.numpy as jnp
from jax import lax
from jax.experimental import pallas as pl
from jax.experimental.pallas import tpu as pltpu
from jax.experimental.pallas import tpu_sc as plsc

_NC = 2
_NS = 16
_NW = _NC * _NS
_L = 16


def _softplus_body(x_ref, o_ref):
    o_ref[...] = jax.nn.softplus(x_ref[...] - 1.0)


@functools.lru_cache(maxsize=None)
def _make_scatter(M, N, D):
    ppw = N // _NW
    blk = min(128, ppw)
    chunks = ppw // blk
    nbuf = 2
    assert chunks % nbuf == 0 and chunks // nbuf >= 2
    mesh = plsc.VectorSubcoreMesh(core_axis_name="c", subcore_axis_name="s",
                                  num_cores=_NC, num_subcores=_NS)

    @functools.partial(
        pl.kernel,
        mesh=mesh,
        compiler_params=pltpu.CompilerParams(needs_layout_passes=False),
        scratch_types=[
            [pltpu.VMEM((blk,), jnp.int32) for _ in range(nbuf)],
            [pltpu.VMEM((1, blk), jnp.float32) for _ in range(nbuf)],
            [pltpu.VMEM((D, blk), jnp.float32) for _ in range(nbuf)],
            [pltpu.VMEM((blk, 128), jnp.float32) for _ in range(nbuf)],
            [pltpu.SemaphoreType.DMA for _ in range(nbuf)],
            [pltpu.SemaphoreType.DMA for _ in range(nbuf)],
        ],
    )
    def scatter(idx_hbm, sp_hbm, vt_hbm, out_hbm,
                idx_vs, sp_vs, slab_vs, rows_vs, in_sems, out_sems):
        wid = lax.axis_index("s") * _NC + lax.axis_index("c")
        base = wid * ppw

        def load_descs(c, j):
            p0 = pl.multiple_of(base + c * blk, blk)
            r0 = p0 // blk
            return (
                pltpu.make_async_copy(idx_hbm.at[pl.ds(p0, blk)], idx_vs[j],
                                      in_sems[j]),
                pltpu.make_async_copy(sp_hbm.at[pl.ds(r0, 1), :], sp_vs[j],
                                      in_sems[j]),
                pltpu.make_async_copy(vt_hbm.at[:, pl.ds(p0, blk)], slab_vs[j],
                                      in_sems[j]),
            )

        def start_load(c, j):
            for d in load_descs(c, j):
                d.start()

        def wait_load(c, j):
            for d in load_descs(c, j):
                d.wait()

        def scatter_desc(j):
            return pltpu.make_async_copy(rows_vs[j], out_hbm.at[idx_vs[j]],
                                         out_sems[j])

        def assemble(j):
            rows_v = rows_vs[j]
            slab_v = slab_vs[j]
            sp_v = sp_vs[j]
            lanes = lax.iota(jnp.int32, _L)
            for g in range(blk // _L):
                plsc.store_scatter(
                    rows_v, [lanes + g * _L, jnp.zeros((_L,), jnp.int32)],
                    sp_v[0, pl.ds(g * _L, _L)])
            def shift(s, carry):
                svec = lax.rem(lanes + s, jnp.int32(_L))
                for c0 in range(0, D, _L):
                    cvec = svec + c0
                    for p0 in range(0, blk, _L):
                        pvec = lanes + p0
                        v = plsc.load_gather(slab_v, [cvec, pvec])
                        plsc.store_scatter(rows_v, [pvec, cvec + 1], v)
                return carry

            lax.fori_loop(0, _L, shift, 0)

        for j in range(nbuf):
            start_load(j, j)

        def group(g, carry):
            c0 = g * nbuf
            for j in range(nbuf):
                wait_load(c0 + j, j)
                assemble(j)
                scatter_desc(j).start()
            nxt = c0 + nbuf

            @pl.when(nxt < chunks)
            def _():
                for j in range(nbuf):
                    scatter_desc(j).wait()
                    start_load(nxt + j, j)

            return carry

        lax.fori_loop(0, chunks // nbuf, group, 0)
        for j in range(nbuf):
            scatter_desc(j).wait()

    return scatter


def kernel(mem, idx, shape_raw, val):
    M, D = mem.shape
    N = idx.shape[0]
    sp2d = pl.pallas_call(
        _softplus_body,
        out_shape=jax.ShapeDtypeStruct((N // 128, 128), jnp.float32),
    )(shape_raw.reshape(N // 128, 128))
    vt = val.T
    out_ref = jax.new_ref(jnp.zeros((M, 128), jnp.float32))
    _make_scatter(M, N, D)(idx, sp2d, vt, out_ref)
    return out_ref[...][:, : D + 1]


# speedup vs baseline: 1.4604x; 1.0331x over previous
import functools

import jax
import jax.numpy as jnp
from jax import lax
from jax.experimental import pallas as pl
from jax.experimental.pallas import tpu as pltpu
from jax.experimental.pallas import tpu_sc as plsc

_NC = 2
_NS = 16
_NW = _NC * _NS
_L = 16


def _softplus_body(x_ref, o_ref):
    o_ref[...] = jax.nn.softplus(x_ref[...] - 1.0)


@functools.lru_cache(maxsize=None)
def _make_scatter(M, N, D):
    ppw = N // _NW
    blk = min(128, ppw)
    chunks = ppw // blk
    nbuf = 4
    assert chunks % nbuf == 0 and chunks // nbuf >= 2
    mesh = plsc.VectorSubcoreMesh(core_axis_name="c", subcore_axis_name="s",
                                  num_cores=_NC, num_subcores=_NS)

    @functools.partial(
        pl.kernel,
        mesh=mesh,
        compiler_params=pltpu.CompilerParams(needs_layout_passes=False),
        scratch_types=[
            [pltpu.VMEM((blk,), jnp.int32) for _ in range(nbuf)],
            [pltpu.VMEM((1, blk), jnp.float32) for _ in range(nbuf)],
            [pltpu.VMEM((D, blk), jnp.float32) for _ in range(nbuf)],
            [pltpu.VMEM((blk, 128), jnp.float32) for _ in range(nbuf)],
            [pltpu.SemaphoreType.DMA for _ in range(nbuf)],
            [pltpu.SemaphoreType.DMA for _ in range(nbuf)],
        ],
    )
    def scatter(idx_hbm, sp_hbm, vt_hbm, out_hbm,
                idx_vs, sp_vs, slab_vs, rows_vs, in_sems, out_sems):
        wid = lax.axis_index("s") * _NC + lax.axis_index("c")
        base = wid * ppw

        def load_descs(c, j):
            p0 = pl.multiple_of(base + c * blk, blk)
            r0 = p0 // blk
            return (
                pltpu.make_async_copy(idx_hbm.at[pl.ds(p0, blk)], idx_vs[j],
                                      in_sems[j]),
                pltpu.make_async_copy(sp_hbm.at[pl.ds(r0, 1), :], sp_vs[j],
                                      in_sems[j]),
                pltpu.make_async_copy(vt_hbm.at[:, pl.ds(p0, blk)], slab_vs[j],
                                      in_sems[j]),
            )

        def start_load(c, j):
            for d in load_descs(c, j):
                d.start()

        def wait_load(c, j):
            for d in load_descs(c, j):
                d.wait()

        def scatter_desc(j):
            return pltpu.make_async_copy(rows_vs[j], out_hbm.at[idx_vs[j]],
                                         out_sems[j])

        def assemble(j):
            rows_v = rows_vs[j]
            slab_v = slab_vs[j]
            sp_v = sp_vs[j]
            lanes = lax.iota(jnp.int32, _L)
            for g in range(blk // _L):
                plsc.store_scatter(
                    rows_v, [lanes + g * _L, jnp.zeros((_L,), jnp.int32)],
                    sp_v[0, pl.ds(g * _L, _L)])
            def shift(s, carry):
                svec = lax.rem(lanes + s, jnp.int32(_L))
                for c0 in range(0, D, _L):
                    cvec = svec + c0
                    for p0 in range(0, blk, _L):
                        pvec = lanes + p0
                        v = plsc.load_gather(slab_v, [cvec, pvec])
                        plsc.store_scatter(rows_v, [pvec, cvec + 1], v)
                return carry

            lax.fori_loop(0, _L, shift, 0)

        for j in range(nbuf):
            start_load(j, j)

        def group(g, carry):
            c0 = g * nbuf
            for j in range(nbuf):
                wait_load(c0 + j, j)
                assemble(j)
                scatter_desc(j).start()
            nxt = c0 + nbuf

            @pl.when(nxt < chunks)
            def _():
                for j in range(nbuf):
                    scatter_desc(j).wait()
                    start_load(nxt + j, j)

            return carry

        lax.fori_loop(0, chunks // nbuf, group, 0)
        for j in range(nbuf):
            scatter_desc(j).wait()

    return scatter


def kernel(mem, idx, shape_raw, val):
    M, D = mem.shape
    N = idx.shape[0]
    sp2d = pl.pallas_call(
        _softplus_body,
        out_shape=jax.ShapeDtypeStruct((N // 128, 128), jnp.float32),
    )(shape_raw.reshape(N // 128, 128))
    vt = val.T
    out_ref = jax.new_ref(jnp.zeros((M, 128), jnp.float32))
    _make_scatter(M, N, D)(idx, sp2d, vt, out_ref)
    return out_ref[...][:, : D + 1]


# speedup vs baseline: 1.4636x; 1.0022x over previous
import functools

import jax
import jax.numpy as jnp
from jax import lax
from jax.experimental import pallas as pl
from jax.experimental.pallas import tpu as pltpu
from jax.experimental.pallas import tpu_sc as plsc

_NC = 2
_NS = 16
_NW = _NC * _NS
_L = 16


def _softplus_body(x_ref, o_ref):
    o_ref[...] = jax.nn.softplus(x_ref[...] - 1.0)


@functools.lru_cache(maxsize=None)
def _make_scatter(M, N, D):
    ppw = N // _NW
    blk = min(128, ppw)
    chunks = ppw // blk
    nbuf = 4
    assert chunks % nbuf == 0 and chunks // nbuf >= 2
    mesh = plsc.VectorSubcoreMesh(core_axis_name="c", subcore_axis_name="s",
                                  num_cores=_NC, num_subcores=_NS)

    @functools.partial(
        pl.kernel,
        mesh=mesh,
        compiler_params=pltpu.CompilerParams(needs_layout_passes=False),
        scratch_types=[
            pltpu.VMEM((chunks, blk), jnp.int32),
            pltpu.VMEM((chunks, blk), jnp.float32),
            [pltpu.VMEM((D, blk), jnp.float32) for _ in range(nbuf)],
            [pltpu.VMEM((blk, 128), jnp.float32) for _ in range(nbuf)],
            pltpu.SemaphoreType.DMA,
            [pltpu.SemaphoreType.DMA for _ in range(nbuf)],
            [pltpu.SemaphoreType.DMA for _ in range(nbuf)],
        ],
    )
    def scatter(idx_hbm, sp_hbm, vt_hbm, out_hbm,
                idx_all, sp_all, slab_vs, rows_vs, pre_sem, in_sems, out_sems):
        wid = lax.axis_index("s") * _NC + lax.axis_index("c")
        base = wid * ppw
        row0 = wid * chunks

        pre = (
            pltpu.make_async_copy(idx_hbm.at[pl.ds(row0, chunks), :],
                                  idx_all, pre_sem),
            pltpu.make_async_copy(sp_hbm.at[pl.ds(row0, chunks), :],
                                  sp_all, pre_sem),
        )
        for d in pre:
            d.start()

        def load_descs(c, j):
            p0 = pl.multiple_of(base + c * blk, blk)
            return (
                pltpu.make_async_copy(vt_hbm.at[:, pl.ds(p0, blk)], slab_vs[j],
                                      in_sems[j]),
            )

        def start_load(c, j):
            for d in load_descs(c, j):
                d.start()

        def wait_load(c, j):
            for d in load_descs(c, j):
                d.wait()

        def scatter_desc(c, j):
            return pltpu.make_async_copy(rows_vs[j],
                                         out_hbm.at[idx_all.at[c]],
                                         out_sems[j])

        def assemble(c, j):
            rows_v = rows_vs[j]
            slab_v = slab_vs[j]
            lanes = lax.iota(jnp.int32, _L)
            for g in range(blk // _L):
                plsc.store_scatter(
                    rows_v, [lanes + g * _L, jnp.zeros((_L,), jnp.int32)],
                    sp_all[c, pl.ds(g * _L, _L)])
            def shift(s, carry):
                svec = lax.rem(lanes + s, jnp.int32(_L))
                for c0 in range(0, D, _L):
                    cvec = svec + c0
                    for p0 in range(0, blk, _L):
                        pvec = lanes + p0
                        v = plsc.load_gather(slab_v, [cvec, pvec])
                        plsc.store_scatter(rows_v, [pvec, cvec + 1], v)
                return carry

            lax.fori_loop(0, _L, shift, 0)

        for j in range(nbuf):
            start_load(j, j)
        for d in pre:
            d.wait()

        def group(g, carry):
            c0 = g * nbuf
            for j in range(nbuf):
                wait_load(c0 + j, j)
                assemble(c0 + j, j)
                scatter_desc(c0 + j, j).start()
            nxt = c0 + nbuf

            @pl.when(nxt < chunks)
            def _():
                for j in range(nbuf):
                    scatter_desc(nxt - nbuf + j, j).wait()
                    start_load(nxt + j, j)

            return carry

        lax.fori_loop(0, chunks // nbuf, group, 0)
        for j in range(nbuf):
            scatter_desc(chunks - nbuf + j, j).wait()

    return scatter


def kernel(mem, idx, shape_raw, val):
    M, D = mem.shape
    N = idx.shape[0]
    sp2d = pl.pallas_call(
        _softplus_body,
        out_shape=jax.ShapeDtypeStruct((N // 128, 128), jnp.float32),
    )(shape_raw.reshape(N // 128, 128))
    vt = val.T
    out_ref = jax.new_ref(jnp.zeros((M, 128), jnp.float32))
    _make_scatter(M, N, D)(idx.reshape(N // 128, 128), sp2d, vt, out_ref)
    return out_ref[...][:, : D + 1]


# speedup vs baseline: 1.5806x; 1.0799x over previous
import functools

import jax
import jax.numpy as jnp
from jax import lax
from jax.experimental import pallas as pl
from jax.experimental.pallas import tpu as pltpu
from jax.experimental.pallas import tpu_sc as plsc

_NC = 2
_NS = 16
_NW = _NC * _NS
_L = 16


def _softplus_body(x_ref, o_ref):
    o_ref[...] = jax.nn.softplus(x_ref[...] - 1.0)


@functools.lru_cache(maxsize=None)
def _make_scatter(M, N, D):
    ppw = N // _NW
    blk = min(128, ppw)
    chunks = ppw // blk
    nbuf = 4
    assert chunks % nbuf == 0 and chunks // nbuf >= 2
    mesh = plsc.VectorSubcoreMesh(core_axis_name="c", subcore_axis_name="s",
                                  num_cores=_NC, num_subcores=_NS)

    @functools.partial(
        pl.kernel,
        mesh=mesh,
        compiler_params=pltpu.CompilerParams(needs_layout_passes=False),
        scratch_types=[
            pltpu.VMEM((chunks, blk), jnp.int32),
            pltpu.VMEM((chunks, blk), jnp.float32),
            [pltpu.VMEM((D, blk), jnp.float32) for _ in range(nbuf)],
            [pltpu.VMEM((blk, 128), jnp.float32) for _ in range(nbuf)],
            pltpu.SemaphoreType.DMA,
            [pltpu.SemaphoreType.DMA for _ in range(nbuf)],
            [pltpu.SemaphoreType.DMA for _ in range(nbuf)],
        ],
    )
    def scatter(idx_hbm, sp_hbm, vt_hbm, out_hbm,
                idx_all, sp_all, slab_vs, rows_vs, pre_sem, in_sems, out_sems):
        wid = lax.axis_index("s") * _NC + lax.axis_index("c")
        base = wid * ppw
        row0 = wid * chunks

        pre = (
            pltpu.make_async_copy(idx_hbm.at[pl.ds(row0, chunks), :],
                                  idx_all, pre_sem),
            pltpu.make_async_copy(sp_hbm.at[pl.ds(row0, chunks), :],
                                  sp_all, pre_sem),
        )
        for d in pre:
            d.start()

        def load_descs(c, j):
            p0 = pl.multiple_of(base + c * blk, blk)
            return (
                pltpu.make_async_copy(vt_hbm.at[:, pl.ds(p0, blk)], slab_vs[j],
                                      in_sems[j]),
            )

        def start_load(c, j):
            for d in load_descs(c, j):
                d.start()

        def wait_load(c, j):
            for d in load_descs(c, j):
                d.wait()

        def scatter_desc(c, j):
            return pltpu.make_async_copy(rows_vs[j],
                                         out_hbm.at[idx_all.at[c]],
                                         out_sems[j])

        def assemble(c, j):
            rows_v = rows_vs[j]
            slab_v = slab_vs[j]
            lanes = lax.iota(jnp.int32, _L)
            for g in range(blk // _L):
                plsc.store_scatter(
                    rows_v, [lanes + g * _L, jnp.zeros((_L,), jnp.int32)],
                    sp_all[c, pl.ds(g * _L, _L)])
            def shift(s, carry):
                svec = lax.rem(lanes + s, jnp.int32(_L))
                for c0 in range(0, D, _L):
                    cvec = svec + c0
                    for p0 in range(0, blk, _L):
                        pvec = lanes + p0
                        v = plsc.load_gather(slab_v, [cvec, pvec])
                        plsc.store_scatter(rows_v, [pvec, cvec + 1], v)
                return carry

            lax.fori_loop(0, _L, shift, 0)

        for j in range(nbuf):
            start_load(j, j)
        for d in pre:
            d.wait()

        def group(g, carry):
            c0 = g * nbuf
            for j in range(nbuf):
                wait_load(c0 + j, j)

                @pl.when(g > 0)
                def _():
                    scatter_desc(c0 - nbuf + j, j).wait()

                assemble(c0 + j, j)
                scatter_desc(c0 + j, j).start()

                @pl.when(c0 + j + nbuf < chunks)
                def _():
                    start_load(c0 + j + nbuf, j)

            return carry

        lax.fori_loop(0, chunks // nbuf, group, 0)
        for j in range(nbuf):
            scatter_desc(chunks - nbuf + j, j).wait()

    return scatter


def kernel(mem, idx, shape_raw, val):
    M, D = mem.shape
    N = idx.shape[0]
    sp2d = pl.pallas_call(
        _softplus_body,
        out_shape=jax.ShapeDtypeStruct((N // 128, 128), jnp.float32),
    )(shape_raw.reshape(N // 128, 128))
    vt = val.T
    out_ref = jax.new_ref(jnp.zeros((M, 128), jnp.float32))
    _make_scatter(M, N, D)(idx.reshape(N // 128, 128), sp2d, vt, out_ref)
    return out_ref[...][:, : D + 1]
